# baseline (speedup 1.0000x reference)
import jax
import jax.numpy as jnp
from jax import lax
from jax.experimental import pallas as pl
from jax.experimental.pallas import tpu as pltpu
from jax.experimental.pallas import tpu_sc as plsc

_L = 16
_NW = 32


def _rsqrt(x):
    i = lax.bitcast_convert_type(x, jnp.int32)
    i = jnp.int32(0x5F3759DF) - lax.shift_right_logical(i, 1)
    y = lax.bitcast_convert_type(i, jnp.float32)
    for _ in range(3):
        y = y * (jnp.float32(1.5) - jnp.float32(0.5) * x * y * y)
    return y


def _lane_sum(x):
    idx = lax.iota(jnp.int32, _L)
    dnums = lax.GatherDimensionNumbers(
        offset_dims=(), collapsed_slice_dims=(0,), start_index_map=(0,))
    for k in (8, 4, 2, 1):
        perm = lax.bitwise_xor(idx, jnp.int32(k))
        x = x + lax.gather(x, perm[:, None], dimension_numbers=dnums,
                           slice_sizes=(1,),
                           mode=lax.GatherScatterMode.PROMISE_IN_BOUNDS)
    return x


def _body(ids_hbm, tt_hbm, tok_hbm, pos_hbm, seg_hbm, gam_hbm, bet_hbm,
          out_hbm, idx_v, tt_v, rows_v, seg_v, pos_v, gam_v, bet_v, sem):
    n_rows, h = ids_hbm.shape
    t_per = (n_rows * h) // _NW
    idx_rows = t_per // h
    s_per_row = pos_hbm.shape[0]
    chunks_per_seq = s_per_row // t_per

    wid = lax.axis_index("s") * 2 + lax.axis_index("c")
    base = wid * t_per
    pos_base = lax.rem(wid, chunks_per_seq) * t_per

    pltpu.sync_copy(ids_hbm.at[pl.ds(wid * idx_rows, idx_rows)], idx_v)
    pltpu.sync_copy(tt_hbm.at[pl.ds(wid * idx_rows, idx_rows)], tt_v)
    pltpu.sync_copy(pos_hbm.at[pl.ds(pos_base, t_per)], pos_v)
    pltpu.sync_copy(gam_hbm, gam_v)
    pltpu.sync_copy(bet_hbm, bet_v)

    copies = []
    for k in range(idx_rows):
        copies.append(pltpu.async_copy(
            tok_hbm.at[idx_v.at[k]], rows_v.at[pl.ds(k * h, h)], sem))
        copies.append(pltpu.async_copy(
            seg_hbm.at[tt_v.at[k]], seg_v.at[pl.ds(k * h, h)], sem))
    for c in copies:
        c.wait()

    nh = h // _L
    gvs = [gam_v[pl.ds(i * _L, _L)] for i in range(nh)]
    bvs = [bet_v[pl.ds(i * _L, _L)] for i in range(nh)]
    inv_h = jnp.float32(1.0 / h)

    def token_body(t, carry):
        es = []
        s1 = jnp.zeros((_L,), jnp.float32)
        s2 = jnp.zeros((_L,), jnp.float32)
        for i in range(nh):
            e = (rows_v[t, pl.ds(i * _L, _L)]
                 + pos_v[t, pl.ds(i * _L, _L)]
                 + seg_v[t, pl.ds(i * _L, _L)])
            es.append(e)
            s1 = s1 + e
            s2 = s2 + e * e
        mean = _lane_sum(s1) * inv_h
        msq = _lane_sum(s2) * inv_h
        var = msq - mean * mean
        r = _rsqrt(var + jnp.float32(1e-5))
        for i in range(nh):
            rows_v[t, pl.ds(i * _L, _L)] = (es[i] - mean) * r * gvs[i] + bvs[i]
        return carry

    lax.fori_loop(0, t_per, token_body, 0)

    pltpu.sync_copy(rows_v, out_hbm.at[pl.ds(base, t_per)])


def kernel(input_ids, token_type_ids, tok_table, pos_table, seg_table,
           gamma, beta):
    b, s = input_ids.shape
    v, h = tok_table.shape
    n = b * s
    t_per = n // _NW
    ids2d = input_ids.reshape(n // h, h).astype(jnp.int32)
    tt2d = token_type_ids.reshape(n // h, h).astype(jnp.int32)

    mesh = plsc.VectorSubcoreMesh(core_axis_name="c", subcore_axis_name="s")
    out = pl.kernel(
        _body,
        out_type=jax.ShapeDtypeStruct((n, h), jnp.float32),
        mesh=mesh,
        scratch_types=[
            pltpu.VMEM((t_per // h, h), jnp.int32),
            pltpu.VMEM((t_per // h, h), jnp.int32),
            pltpu.VMEM((t_per, h), jnp.float32),
            pltpu.VMEM((t_per, h), jnp.float32),
            pltpu.VMEM((t_per, h), jnp.float32),
            pltpu.VMEM((h,), jnp.float32),
            pltpu.VMEM((h,), jnp.float32),
            pltpu.SemaphoreType.DMA,
        ],
    )(ids2d, tt2d, tok_table, pos_table, seg_table, gamma, beta)
    return out.reshape(b, s, h)


# speedup vs baseline: 4.7978x; 4.7978x over previous
import jax
import jax.numpy as jnp
from jax import lax
from jax.experimental import pallas as pl
from jax.experimental.pallas import tpu as pltpu
from jax.experimental.pallas import tpu_sc as plsc

_L = 16
_NW = 32


def _rsqrt(x):
    i = lax.bitcast_convert_type(x, jnp.int32)
    i = jnp.int32(0x5F3759DF) - lax.shift_right_logical(i, 1)
    y = lax.bitcast_convert_type(i, jnp.float32)
    for _ in range(3):
        y = y * (jnp.float32(1.5) - jnp.float32(0.5) * x * y * y)
    return y


def _lane_sum(x):
    idx = lax.iota(jnp.int32, _L)
    dnums = lax.GatherDimensionNumbers(
        offset_dims=(), collapsed_slice_dims=(0,), start_index_map=(0,))
    for k in (8, 4, 2, 1):
        perm = lax.bitwise_xor(idx, jnp.int32(k))
        x = x + lax.gather(x, perm[:, None], dimension_numbers=dnums,
                           slice_sizes=(1,),
                           mode=lax.GatherScatterMode.PROMISE_IN_BOUNDS)
    return x


def _make_body(seq_len):
    def _body(ids_hbm, tt_hbm, tok_hbm, ps_hbm, gam_hbm, bet_hbm,
              out_hbm, idx_v, tt_v, psi_v, rows_v, ps_v, gam_v, bet_v, sem):
        n_rows, h = ids_hbm.shape
        t_per = (n_rows * h) // _NW
        idx_rows = t_per // h
        chunks_per_seq = seq_len // t_per

        wid = lax.axis_index("s") * 2 + lax.axis_index("c")
        base = wid * t_per
        pos_base = lax.rem(wid, chunks_per_seq) * t_per

        cp_ids = pltpu.async_copy(
            ids_hbm.at[pl.ds(wid * idx_rows, idx_rows)], idx_v, sem)
        cp_tt = pltpu.async_copy(
            tt_hbm.at[pl.ds(wid * idx_rows, idx_rows)], tt_v, sem)
        pltpu.sync_copy(gam_hbm, gam_v)
        pltpu.sync_copy(bet_hbm, bet_v)
        cp_ids.wait()
        cp_tt.wait()

        lanes = lax.iota(jnp.int32, _L)
        for k in range(idx_rows):
            for j in range(h // _L):
                tt16 = tt_v[k, pl.ds(j * _L, _L)]
                pos16 = pos_base + (k * h + j * _L) + lanes
                psi_v[k, pl.ds(j * _L, _L)] = tt16 * seq_len + pos16

        copies = []
        for k in range(idx_rows):
            copies.append(pltpu.async_copy(
                tok_hbm.at[idx_v.at[k]], rows_v.at[pl.ds(k * h, h)], sem))
            copies.append(pltpu.async_copy(
                ps_hbm.at[psi_v.at[k]], ps_v.at[pl.ds(k * h, h)], sem))
        for c in copies:
            c.wait()

        nh = h // _L
        gvs = [gam_v[pl.ds(i * _L, _L)] for i in range(nh)]
        bvs = [bet_v[pl.ds(i * _L, _L)] for i in range(nh)]
        inv_h = jnp.float32(1.0 / h)

        def token_body(t, carry):
            es = []
            s1 = jnp.zeros((_L,), jnp.float32)
            s2 = jnp.zeros((_L,), jnp.float32)
            for i in range(nh):
                e = rows_v[t, pl.ds(i * _L, _L)] + ps_v[t, pl.ds(i * _L, _L)]
                es.append(e)
                s1 = s1 + e
                s2 = s2 + e * e
            mean = _lane_sum(s1) * inv_h
            msq = _lane_sum(s2) * inv_h
            var = msq - mean * mean
            r = _rsqrt(var + jnp.float32(1e-5))
            for i in range(nh):
                rg = r * gvs[i]
                rows_v[t, pl.ds(i * _L, _L)] = (es[i] - mean) * rg + bvs[i]
            return carry

        lax.fori_loop(0, t_per, token_body, 0)

        pltpu.sync_copy(rows_v, out_hbm.at[pl.ds(base, t_per)])

    return _body


def kernel(input_ids, token_type_ids, tok_table, pos_table, seg_table,
           gamma, beta):
    b, s = input_ids.shape
    v, h = tok_table.shape
    n = b * s
    t_per = n // _NW
    ids2d = input_ids.reshape(n // h, h).astype(jnp.int32)
    tt2d = token_type_ids.reshape(n // h, h).astype(jnp.int32)
    ps_table = (pos_table[None, :, :] + seg_table[:, None, :]).reshape(-1, h)

    mesh = plsc.VectorSubcoreMesh(core_axis_name="c", subcore_axis_name="s")
    out = pl.kernel(
        _make_body(s),
        out_type=jax.ShapeDtypeStruct((n, h), jnp.float32),
        mesh=mesh,
        scratch_types=[
            pltpu.VMEM((t_per // h, h), jnp.int32),
            pltpu.VMEM((t_per // h, h), jnp.int32),
            pltpu.VMEM((t_per // h, h), jnp.int32),
            pltpu.VMEM((t_per, h), jnp.float32),
            pltpu.VMEM((t_per, h), jnp.float32),
            pltpu.VMEM((h,), jnp.float32),
            pltpu.VMEM((h,), jnp.float32),
            pltpu.SemaphoreType.DMA,
        ],
    )(ids2d, tt2d, tok_table, ps_table, gamma, beta)
    return out.reshape(b, s, h)


# speedup vs baseline: 5.1134x; 1.0658x over previous
import jax
import jax.numpy as jnp
from jax import lax
from jax.experimental import pallas as pl
from jax.experimental.pallas import tpu as pltpu
from jax.experimental.pallas import tpu_sc as plsc

_L = 16
_NW = 32


def _rsqrt(x):
    i = lax.bitcast_convert_type(x, jnp.int32)
    i = jnp.int32(0x5F3759DF) - lax.shift_right_logical(i, 1)
    y = lax.bitcast_convert_type(i, jnp.float32)
    for _ in range(3):
        y = y * (jnp.float32(1.5) - jnp.float32(0.5) * x * y * y)
    return y


def _lane_sum(x):
    idx = lax.iota(jnp.int32, _L)
    dnums = lax.GatherDimensionNumbers(
        offset_dims=(), collapsed_slice_dims=(0,), start_index_map=(0,))
    for k in (8, 4, 2, 1):
        perm = lax.bitwise_xor(idx, jnp.int32(k))
        x = x + lax.gather(x, perm[:, None], dimension_numbers=dnums,
                           slice_sizes=(1,),
                           mode=lax.GatherScatterMode.PROMISE_IN_BOUNDS)
    return x


def _body(ids_hbm, tt_hbm, tok_hbm, ps_hbm, gam_hbm, bet_hbm,
          out_hbm, idx_v, tt_v, psi_v, rows_v, ps_v, gam_v, bet_v,
          sem, osem, *gsems):
    b, seq_len = ids_hbm.shape
    h = tok_hbm.shape[1]
    t_per = (b * seq_len) // _NW
    n_grp = t_per // h
    chunks_per_seq = seq_len // t_per

    wid = lax.axis_index("s") * 2 + lax.axis_index("c")
    bi = wid // chunks_per_seq
    s0 = lax.rem(wid, chunks_per_seq) * t_per

    cp_ids = pltpu.async_copy(ids_hbm.at[bi, pl.ds(s0, t_per)], idx_v, sem)
    cp_tt = pltpu.async_copy(tt_hbm.at[bi, pl.ds(s0, t_per)], tt_v, sem)
    pltpu.sync_copy(gam_hbm, gam_v)
    pltpu.sync_copy(bet_hbm, bet_v)
    cp_ids.wait()
    cp_tt.wait()

    lanes = lax.iota(jnp.int32, _L)
    for g in range(t_per // _L):
        tt16 = tt_v[pl.ds(g * _L, _L)]
        psi_v[pl.ds(g * _L, _L)] = tt16 * seq_len + (s0 + g * _L) + lanes

    copies = []
    for k in range(n_grp):
        copies.append(pltpu.async_copy(
            tok_hbm.at[idx_v.at[pl.ds(k * h, h)]],
            rows_v.at[pl.ds(k * h, h)], gsems[k]))
        copies.append(pltpu.async_copy(
            ps_hbm.at[psi_v.at[pl.ds(k * h, h)]],
            ps_v.at[pl.ds(k * h, h)], gsems[k]))

    nh = h // _L
    gvs = [gam_v[pl.ds(i * _L, _L)] for i in range(nh)]
    bvs = [bet_v[pl.ds(i * _L, _L)] for i in range(nh)]
    inv_h = jnp.float32(1.0 / h)

    def token_body(t, carry):
        es = []
        s1 = jnp.zeros((_L,), jnp.float32)
        s2 = jnp.zeros((_L,), jnp.float32)
        for i in range(nh):
            e = rows_v[t, pl.ds(i * _L, _L)] + ps_v[t, pl.ds(i * _L, _L)]
            es.append(e)
            s1 = s1 + e
            s2 = s2 + e * e
        mean = _lane_sum(s1) * inv_h
        msq = _lane_sum(s2) * inv_h
        var = msq - mean * mean
        r = _rsqrt(var + jnp.float32(1e-5))
        for i in range(nh):
            rg = r * gvs[i]
            rows_v[t, pl.ds(i * _L, _L)] = (es[i] - mean) * rg + bvs[i]
        return carry

    out_cps = []
    for k in range(n_grp):
        copies[2 * k].wait()
        copies[2 * k + 1].wait()
        lax.fori_loop(k * h, (k + 1) * h, token_body, 0)
        out_cps.append(pltpu.async_copy(
            rows_v.at[pl.ds(k * h, h)],
            out_hbm.at[bi, pl.ds(s0 + k * h, h)], osem))
    for c in out_cps:
        c.wait()


def kernel(input_ids, token_type_ids, tok_table, pos_table, seg_table,
           gamma, beta):
    b, s = input_ids.shape
    v, h = tok_table.shape
    n = b * s
    t_per = n // _NW
    ids = input_ids.astype(jnp.int32)
    tt = token_type_ids.astype(jnp.int32)
    ps_table = (pos_table[None, :, :] + seg_table[:, None, :]).reshape(-1, h)

    mesh = plsc.VectorSubcoreMesh(core_axis_name="c", subcore_axis_name="s")
    out = pl.kernel(
        _body,
        out_type=jax.ShapeDtypeStruct((b, s, h), jnp.float32),
        mesh=mesh,
        scratch_types=[
            pltpu.VMEM((t_per,), jnp.int32),
            pltpu.VMEM((t_per,), jnp.int32),
            pltpu.VMEM((t_per,), jnp.int32),
            pltpu.VMEM((t_per, h), jnp.float32),
            pltpu.VMEM((t_per, h), jnp.float32),
            pltpu.VMEM((h,), jnp.float32),
            pltpu.VMEM((h,), jnp.float32),
            pltpu.SemaphoreType.DMA,
            pltpu.SemaphoreType.DMA,
        ] + [pltpu.SemaphoreType.DMA] * (t_per // h),
    )(ids, tt, tok_table, ps_table, gamma, beta)
    return out


# speedup vs baseline: 5.9139x; 1.1565x over previous
import jax
import jax.numpy as jnp
from jax import lax
from jax.experimental import pallas as pl
from jax.experimental.pallas import tpu as pltpu
from jax.experimental.pallas import tpu_sc as plsc

_L = 16
_NW = 32


def _rsqrt(x):
    i = lax.bitcast_convert_type(x, jnp.int32)
    i = jnp.int32(0x5F3759DF) - lax.shift_right_logical(i, 1)
    y = lax.bitcast_convert_type(i, jnp.float32)
    for _ in range(2):
        y = y * (jnp.float32(1.5) - jnp.float32(0.5) * x * y * y)
    return y


def _lane_sum(x):
    idx = lax.iota(jnp.int32, _L)
    dnums = lax.GatherDimensionNumbers(
        offset_dims=(), collapsed_slice_dims=(0,), start_index_map=(0,))
    for k in (8, 4, 2, 1):
        perm = lax.bitwise_xor(idx, jnp.int32(k))
        x = x + lax.gather(x, perm[:, None], dimension_numbers=dnums,
                           slice_sizes=(1,),
                           mode=lax.GatherScatterMode.PROMISE_IN_BOUNDS)
    return x


def _body(ids_hbm, tt_hbm, tok_hbm, ps_hbm, gam_hbm, bet_hbm,
          out_hbm, idx_v, tt_v, psi_v, rows_v, ps_v, gam_v, bet_v,
          sem, osem, *gsems):
    b, seq_len = ids_hbm.shape
    h = tok_hbm.shape[1]
    t_per = (b * seq_len) // _NW
    n_grp = t_per // h
    chunks_per_seq = seq_len // t_per

    wid = lax.axis_index("s") * 2 + lax.axis_index("c")
    bi = wid // chunks_per_seq
    s0 = lax.rem(wid, chunks_per_seq) * t_per

    cp_ids = pltpu.async_copy(ids_hbm.at[bi, pl.ds(s0, t_per)], idx_v, sem)
    cp_tt = pltpu.async_copy(tt_hbm.at[bi, pl.ds(s0, t_per)], tt_v, sem)
    pltpu.sync_copy(gam_hbm, gam_v)
    pltpu.sync_copy(bet_hbm, bet_v)
    cp_ids.wait()
    cp_tt.wait()

    lanes = lax.iota(jnp.int32, _L)
    for g in range(t_per // _L):
        tt16 = tt_v[pl.ds(g * _L, _L)]
        psi_v[pl.ds(g * _L, _L)] = tt16 * seq_len + (s0 + g * _L) + lanes

    copies = []
    for k in range(n_grp):
        copies.append(pltpu.async_copy(
            tok_hbm.at[idx_v.at[pl.ds(k * h, h)]],
            rows_v.at[pl.ds(k * h, h)], gsems[k]))
        copies.append(pltpu.async_copy(
            ps_hbm.at[psi_v.at[pl.ds(k * h, h)]],
            ps_v.at[pl.ds(k * h, h)], gsems[k]))

    nh = h // _L
    gvs = [gam_v[pl.ds(i * _L, _L)] for i in range(nh)]
    bvs = [bet_v[pl.ds(i * _L, _L)] for i in range(nh)]
    inv_h = jnp.float32(1.0 / h)

    def one_token(t):
        es = []
        s1 = jnp.zeros((_L,), jnp.float32)
        s2 = jnp.zeros((_L,), jnp.float32)
        for i in range(nh):
            e = rows_v[t, pl.ds(i * _L, _L)] + ps_v[t, pl.ds(i * _L, _L)]
            es.append(e)
            s1 = s1 + e
            s2 = s2 + e * e
        return es, s1, s2

    def finish_token(t, es, s1, s2):
        mean = _lane_sum(s1) * inv_h
        msq = _lane_sum(s2) * inv_h
        var = msq - mean * mean
        r = _rsqrt(var + jnp.float32(1e-5))
        for i in range(nh):
            rg = r * gvs[i]
            rows_v[t, pl.ds(i * _L, _L)] = (es[i] - mean) * rg + bvs[i]

    def pair_body(p, carry):
        t0 = p * 2
        t1 = t0 + 1
        es0, s10, s20 = one_token(t0)
        es1, s11, s21 = one_token(t1)
        finish_token(t0, es0, s10, s20)
        finish_token(t1, es1, s11, s21)
        return carry

    out_cps = []
    for k in range(n_grp):
        copies[2 * k].wait()
        copies[2 * k + 1].wait()
        lax.fori_loop(k * h // 2, (k + 1) * h // 2, pair_body, 0)
        out_cps.append(pltpu.async_copy(
            rows_v.at[pl.ds(k * h, h)],
            out_hbm.at[bi, pl.ds(s0 + k * h, h)], osem))
    for c in out_cps:
        c.wait()


def kernel(input_ids, token_type_ids, tok_table, pos_table, seg_table,
           gamma, beta):
    b, s = input_ids.shape
    v, h = tok_table.shape
    n = b * s
    t_per = n // _NW
    ids = input_ids.astype(jnp.int32)
    tt = token_type_ids.astype(jnp.int32)
    ps_table = (pos_table[None, :, :] + seg_table[:, None, :]).reshape(-1, h)

    mesh = plsc.VectorSubcoreMesh(core_axis_name="c", subcore_axis_name="s")
    out = pl.kernel(
        _body,
        out_type=jax.ShapeDtypeStruct((b, s, h), jnp.float32),
        mesh=mesh,
        scratch_types=[
            pltpu.VMEM((t_per,), jnp.int32),
            pltpu.VMEM((t_per,), jnp.int32),
            pltpu.VMEM((t_per,), jnp.int32),
            pltpu.VMEM((t_per, h), jnp.float32),
            pltpu.VMEM((t_per, h), jnp.float32),
            pltpu.VMEM((h,), jnp.float32),
            pltpu.VMEM((h,), jnp.float32),
            pltpu.SemaphoreType.DMA,
            pltpu.SemaphoreType.DMA,
        ] + [pltpu.SemaphoreType.DMA] * (t_per // h),
    )(ids, tt, tok_table, ps_table, gamma, beta)
    return out
